# baseline (speedup 1.0000x reference)
import functools

import jax
import jax.numpy as jnp
from jax.experimental import pallas as pl
from jax.experimental.pallas import tpu as pltpu


def _body(td_ref, mask_ref, lae_ref, ltm_ref, ll_ref,
          ev_ref, ne_ref, cnt_ref,
          acc_ne, acc_ev, acc_cnt, *, inv_n, rows):
    i = pl.program_id(0)

    @pl.when(i == 0)
    def _init():
        acc_ne[...] = jnp.zeros_like(acc_ne)
        acc_ev[...] = jnp.zeros_like(acc_ev)
        acc_cnt[...] = jnp.zeros_like(acc_cnt)

    maskf = mask_ref[...]
    w = td_ref[...] * maskf * inv_n
    t = ll_ref[...] * w
    acc_ne[...] += jnp.sum(t.reshape(rows // 8, 8, t.shape[-1]), axis=0)

    ev_l = jnp.sum(lae_ref[...] * ltm_ref[...], axis=1, keepdims=True)
    ev_t = jnp.log(jnp.where(maskf > 0, ev_l, 1.0))
    acc_ev[...] += jnp.sum(ev_t.reshape(rows // 8, 8, 1), axis=0)
    acc_cnt[...] += jnp.sum(maskf.reshape(rows // 8, 8, 1), axis=0)

    @pl.when(i == pl.num_programs(0) - 1)
    def _fini():
        ne_ref[0, 0] = jnp.sum(acc_ne[...])
        ev_ref[0, 0] = jnp.sum(acc_ev[...])
        cnt_ref[0, 0] = jnp.sum(acc_cnt[...]).astype(jnp.int32)


def kernel(time_delta_seq, lambda_at_event, lambdas_loss_samples, seq_mask, lambda_type_mask):
    B, S, N, K = lambdas_loss_samples.shape
    NK = N * K
    M = B * S
    ll = lambdas_loss_samples.reshape(M, NK)
    td = time_delta_seq.reshape(M, 1)
    maskf = seq_mask.astype(jnp.float32).reshape(M, 1)
    lae = lambda_at_event.reshape(M, K)
    ltm = lambda_type_mask.reshape(M, K)

    R = 1024
    grid = (M // R,)

    body = functools.partial(_body, inv_n=1.0 / N, rows=R)
    ev, ne, cnt = pl.pallas_call(
        body,
        grid=grid,
        in_specs=[
            pl.BlockSpec((R, 1), lambda i: (i, 0)),
            pl.BlockSpec((R, 1), lambda i: (i, 0)),
            pl.BlockSpec((R, K), lambda i: (i, 0)),
            pl.BlockSpec((R, K), lambda i: (i, 0)),
            pl.BlockSpec((R, NK), lambda i: (i, 0)),
        ],
        out_specs=[
            pl.BlockSpec(memory_space=pltpu.SMEM),
            pl.BlockSpec(memory_space=pltpu.SMEM),
            pl.BlockSpec(memory_space=pltpu.SMEM),
        ],
        out_shape=[
            jax.ShapeDtypeStruct((1, 1), jnp.float32),
            jax.ShapeDtypeStruct((1, 1), jnp.float32),
            jax.ShapeDtypeStruct((1, 1), jnp.int32),
        ],
        scratch_shapes=[
            pltpu.VMEM((8, NK), jnp.float32),
            pltpu.VMEM((8, 1), jnp.float32),
            pltpu.VMEM((8, 1), jnp.float32),
        ],
    )(td, maskf, lae, ltm, ll)

    return (ev[0, 0], ne[0, 0], cnt[0, 0])


# speedup vs baseline: 2.0751x; 2.0751x over previous
import functools

import jax
import jax.numpy as jnp
from jax import lax
from jax.experimental import pallas as pl
from jax.experimental.pallas import tpu as pltpu


def _body(td_ref, mask_ref, lae_ref, ltm_ref, ll_ref,
          ev_ref, ne_ref, cnt_ref,
          acc_ne, acc_ev, acc_cnt, *, inv_n):
    i = pl.program_id(0)

    @pl.when(i == 0)
    def _init():
        acc_ne[...] = jnp.zeros_like(acc_ne)
        acc_ev[...] = jnp.zeros_like(acc_ev)
        acc_cnt[...] = jnp.zeros_like(acc_cnt)

    maskf = mask_ref[...]
    w = td_ref[...] * maskf * inv_n
    acc_ne[...] += lax.dot_general(
        w, ll_ref[...],
        dimension_numbers=(((1,), (1,)), ((0,), (0,))),
        preferred_element_type=jnp.float32,
    )

    ev_l = jnp.sum(lae_ref[...] * ltm_ref[...], axis=2)
    acc_ev[...] += jnp.log(jnp.where(maskf > 0, ev_l, 1.0))
    acc_cnt[...] += maskf

    @pl.when(i == pl.num_programs(0) - 1)
    def _fini():
        ne_ref[0, 0] = jnp.sum(acc_ne[...])
        ev_ref[0, 0] = jnp.sum(acc_ev[...])
        cnt_ref[0, 0] = jnp.sum(acc_cnt[...]).astype(jnp.int32)


def kernel(time_delta_seq, lambda_at_event, lambdas_loss_samples, seq_mask, lambda_type_mask):
    B, S, N, K = lambdas_loss_samples.shape
    NK = N * K
    ll = lambdas_loss_samples.reshape(B, S, NK)
    maskf = seq_mask.astype(jnp.float32)

    CH = 256
    grid = (S // CH,)

    body = functools.partial(_body, inv_n=1.0 / N)
    ev, ne, cnt = pl.pallas_call(
        body,
        grid=grid,
        in_specs=[
            pl.BlockSpec((B, CH), lambda i: (0, i)),
            pl.BlockSpec((B, CH), lambda i: (0, i)),
            pl.BlockSpec((B, CH, K), lambda i: (0, i, 0)),
            pl.BlockSpec((B, CH, K), lambda i: (0, i, 0)),
            pl.BlockSpec((B, CH, NK), lambda i: (0, i, 0)),
        ],
        out_specs=[
            pl.BlockSpec(memory_space=pltpu.SMEM),
            pl.BlockSpec(memory_space=pltpu.SMEM),
            pl.BlockSpec(memory_space=pltpu.SMEM),
        ],
        out_shape=[
            jax.ShapeDtypeStruct((1, 1), jnp.float32),
            jax.ShapeDtypeStruct((1, 1), jnp.float32),
            jax.ShapeDtypeStruct((1, 1), jnp.int32),
        ],
        scratch_shapes=[
            pltpu.VMEM((B, NK), jnp.float32),
            pltpu.VMEM((B, CH), jnp.float32),
            pltpu.VMEM((B, CH), jnp.float32),
        ],
    )(time_delta_seq, maskf, lambda_at_event, lambda_type_mask, ll)

    return (ev[0, 0], ne[0, 0], cnt[0, 0])
